# baseline (speedup 1.0000x reference)
import functools

import jax
import jax.numpy as jnp
from jax.experimental import pallas as pl
from jax.experimental.pallas import tpu as pltpu

_BM = 400
_NBUF = 2


def _main_kernel(
    adj_hbm, x_hbm, hx_hbm, cx_hbm, g_ref, wx_ref, wh_ref, gb_ref, bias_ref,
    hy_ref, cy_ref,
    adj_buf, x_buf, hx_buf, cx_buf, sup_ref,
    adj_sem, x_sem, hx_sem, cx_sem, *, h: int
):
    n = adj_hbm.shape[0]
    ns = n // _BM

    def adj_copy(s, b):
        return pltpu.make_async_copy(
            adj_hbm.at[pl.ds(s * _BM, _BM), :], adj_buf.at[b], adj_sem.at[b]
        )

    def hx_copy(s, b):
        return pltpu.make_async_copy(
            hx_hbm.at[pl.ds(s * _BM, _BM), :], hx_buf.at[b], hx_sem.at[b]
        )

    def cx_copy(s, b):
        return pltpu.make_async_copy(
            cx_hbm.at[pl.ds(s * _BM, _BM), :], cx_buf.at[b], cx_sem.at[b]
        )

    x_copy = pltpu.make_async_copy(x_hbm, x_buf, x_sem)

    adj_copy(0, 0).start()
    x_copy.start()
    hx_copy(0, 0).start()
    cx_copy(0, 0).start()
    for s in range(1, min(_NBUF, ns)):
        adj_copy(s, s).start()
        hx_copy(s, s).start()
        cx_copy(s, s).start()

    x_copy.wait()
    sup_ref[...] = jnp.dot(
        x_buf[...], g_ref[...], preferred_element_type=jnp.float32
    )

    for s in range(ns):
        b = s % _NBUF
        adj_copy(s, b).wait()
        hx_copy(s, b).wait()
        cx_copy(s, b).wait()

        acc = jnp.dot(
            adj_buf[b], sup_ref[...], preferred_element_type=jnp.float32
        )
        xs = jnp.maximum(acc, 0.0) + bias_ref[...]
        gates = (
            jnp.dot(xs, wx_ref[...], preferred_element_type=jnp.float32)
            + jnp.dot(hx_buf[b], wh_ref[...],
                      preferred_element_type=jnp.float32)
            + gb_ref[...]
        )
        ingate = jax.nn.sigmoid(gates[:, 0:h])
        forgetgate = jax.nn.sigmoid(gates[:, h:2 * h])
        cellgate = jnp.tanh(gates[:, 2 * h:3 * h])
        outgate = jax.nn.sigmoid(gates[:, 3 * h:4 * h])
        cy = cx_buf[b] * forgetgate + ingate * cellgate
        cy_ref[pl.ds(s * _BM, _BM), :] = cy
        hy_ref[pl.ds(s * _BM, _BM), :] = outgate * jnp.tanh(cy)

        nxt = s + _NBUF
        if nxt < ns:
            adj_copy(nxt, b).start()
            hx_copy(nxt, b).start()
            cx_copy(nxt, b).start()


@jax.jit
def kernel(x, hx, cx, adj, gcn_weight, W_x2h, b_x2h, W_h2h, b_h2h, bias):
    n, d = x.shape
    h = hx.shape[1]

    wx_t = W_x2h.T
    wh_t = W_h2h.T
    gate_b = (b_x2h + b_h2h).reshape(1, 4 * h)
    bias2d = bias.reshape(1, h)

    hbm = pl.BlockSpec(memory_space=pltpu.MemorySpace.HBM)
    vmem = pl.BlockSpec(memory_space=pltpu.MemorySpace.VMEM)

    hy, cy = pl.pallas_call(
        functools.partial(_main_kernel, h=h),
        in_specs=[hbm, hbm, hbm, hbm, vmem, vmem, vmem, vmem, vmem],
        out_specs=[vmem, vmem],
        out_shape=[
            jax.ShapeDtypeStruct((n, h), jnp.float32),
            jax.ShapeDtypeStruct((n, h), jnp.float32),
        ],
        scratch_shapes=[
            pltpu.VMEM((_NBUF, _BM, n), jnp.float32),
            pltpu.VMEM((n, d), jnp.float32),
            pltpu.VMEM((_NBUF, _BM, h), jnp.float32),
            pltpu.VMEM((_NBUF, _BM, h), jnp.float32),
            pltpu.VMEM((n, h), jnp.float32),
            pltpu.SemaphoreType.DMA((_NBUF,)),
            pltpu.SemaphoreType.DMA,
            pltpu.SemaphoreType.DMA((_NBUF,)),
            pltpu.SemaphoreType.DMA((_NBUF,)),
        ],
    )(adj, x, hx, cx, gcn_weight, wx_t, wh_t, gate_b, bias2d)

    return (hy, cy)


# speedup vs baseline: 1.0236x; 1.0236x over previous
import functools

import jax
import jax.numpy as jnp
from jax.experimental import pallas as pl
from jax.experimental.pallas import tpu as pltpu

_BM = 400
_NBUF = 3
_NOUT = 2


def _main_kernel(
    adj_hbm, x_hbm, hx_hbm, cx_hbm, g_ref, wx_ref, wh_ref, gb_ref, bias_ref,
    hy_hbm, cy_hbm,
    adj_buf, x_buf, hx_buf, cx_buf, sup_ref, hy_stage, cy_stage,
    adj_sem, x_sem, hx_sem, cx_sem, hy_sem, cy_sem, *, h: int
):
    n = adj_hbm.shape[0]
    ns = n // _BM

    def adj_copy(s, b):
        return pltpu.make_async_copy(
            adj_hbm.at[pl.ds(s * _BM, _BM), :], adj_buf.at[b], adj_sem.at[b]
        )

    def hx_copy(s, b):
        return pltpu.make_async_copy(
            hx_hbm.at[pl.ds(s * _BM, _BM), :], hx_buf.at[b], hx_sem.at[b]
        )

    def cx_copy(s, b):
        return pltpu.make_async_copy(
            cx_hbm.at[pl.ds(s * _BM, _BM), :], cx_buf.at[b], cx_sem.at[b]
        )

    def hy_copy(s, b):
        return pltpu.make_async_copy(
            hy_stage.at[b], hy_hbm.at[pl.ds(s * _BM, _BM), :], hy_sem.at[b]
        )

    def cy_copy(s, b):
        return pltpu.make_async_copy(
            cy_stage.at[b], cy_hbm.at[pl.ds(s * _BM, _BM), :], cy_sem.at[b]
        )

    x_copy = pltpu.make_async_copy(x_hbm, x_buf, x_sem)

    adj_copy(0, 0).start()
    x_copy.start()
    hx_copy(0, 0).start()
    cx_copy(0, 0).start()
    for s in range(1, min(_NBUF - 1, ns)):
        adj_copy(s, s).start()
        hx_copy(s, s).start()
        cx_copy(s, s).start()

    x_copy.wait()
    sup_ref[...] = jnp.dot(
        x_buf[...], g_ref[...], preferred_element_type=jnp.float32
    )

    for s in range(ns):
        b = s % _NBUF
        adj_copy(s, b).wait()
        hx_copy(s, b).wait()
        cx_copy(s, b).wait()

        nxt = s + _NBUF - 1
        if nxt < ns:
            bn = nxt % _NBUF
            adj_copy(nxt, bn).start()
            hx_copy(nxt, bn).start()
            cx_copy(nxt, bn).start()

        acc = jnp.dot(
            adj_buf[b], sup_ref[...], preferred_element_type=jnp.float32
        )
        xs = jnp.maximum(acc, 0.0) + bias_ref[...]
        gates = (
            jnp.dot(xs, wx_ref[...], preferred_element_type=jnp.float32)
            + jnp.dot(hx_buf[b], wh_ref[...],
                      preferred_element_type=jnp.float32)
            + gb_ref[...]
        )
        ingate = jax.nn.sigmoid(gates[:, 0:h])
        forgetgate = jax.nn.sigmoid(gates[:, h:2 * h])
        cellgate = jnp.tanh(gates[:, 2 * h:3 * h])
        outgate = jax.nn.sigmoid(gates[:, 3 * h:4 * h])
        cy = cx_buf[b] * forgetgate + ingate * cellgate
        hy = outgate * jnp.tanh(cy)

        bo = s % _NOUT
        if s >= _NOUT:
            hy_copy(s - _NOUT, bo).wait()
            cy_copy(s - _NOUT, bo).wait()
        hy_stage[bo] = hy
        cy_stage[bo] = cy
        hy_copy(s, bo).start()
        cy_copy(s, bo).start()

    for s in range(max(ns - _NOUT, 0), ns):
        hy_copy(s, s % _NOUT).wait()
        cy_copy(s, s % _NOUT).wait()


@jax.jit
def kernel(x, hx, cx, adj, gcn_weight, W_x2h, b_x2h, W_h2h, b_h2h, bias):
    n, d = x.shape
    h = hx.shape[1]

    wx_t = W_x2h.T
    wh_t = W_h2h.T
    gate_b = (b_x2h + b_h2h).reshape(1, 4 * h)
    bias2d = bias.reshape(1, h)

    hbm = pl.BlockSpec(memory_space=pltpu.MemorySpace.HBM)
    vmem = pl.BlockSpec(memory_space=pltpu.MemorySpace.VMEM)

    hy, cy = pl.pallas_call(
        functools.partial(_main_kernel, h=h),
        in_specs=[hbm, hbm, hbm, hbm, vmem, vmem, vmem, vmem, vmem],
        out_specs=[hbm, hbm],
        out_shape=[
            jax.ShapeDtypeStruct((n, h), jnp.float32),
            jax.ShapeDtypeStruct((n, h), jnp.float32),
        ],
        scratch_shapes=[
            pltpu.VMEM((_NBUF, _BM, n), jnp.float32),
            pltpu.VMEM((n, d), jnp.float32),
            pltpu.VMEM((_NBUF, _BM, h), jnp.float32),
            pltpu.VMEM((_NBUF, _BM, h), jnp.float32),
            pltpu.VMEM((n, h), jnp.float32),
            pltpu.VMEM((_NOUT, _BM, h), jnp.float32),
            pltpu.VMEM((_NOUT, _BM, h), jnp.float32),
            pltpu.SemaphoreType.DMA((_NBUF,)),
            pltpu.SemaphoreType.DMA,
            pltpu.SemaphoreType.DMA((_NBUF,)),
            pltpu.SemaphoreType.DMA((_NBUF,)),
            pltpu.SemaphoreType.DMA((_NOUT,)),
            pltpu.SemaphoreType.DMA((_NOUT,)),
        ],
    )(adj, x, hx, cx, gcn_weight, wx_t, wh_t, gate_b, bias2d)

    return (hy, cy)
